# baseline (speedup 1.0000x reference)
import functools

import jax
import jax.numpy as jnp
from jax import lax
from jax.experimental import pallas as pl
from jax.experimental.pallas import tpu as pltpu
from jax.experimental.pallas import tpu_sc as plsc

_VOCAB = 1000000
_EMB = 64
_NODE = 64
_BATCH = 16384

_info = plsc.get_sparse_core_info()
_NC = _info.num_cores
_NS = _info.num_subcores
_NW = _NC * _NS
_BPW = _BATCH // _NW
_CH = 128
_NCH = _BPW // _CH

_sc_mesh = plsc.VectorSubcoreMesh(core_axis_name="c", subcore_axis_name="s")


@functools.partial(
    pl.kernel,
    mesh=_sc_mesh,
    compiler_params=pltpu.CompilerParams(use_tc_tiling_on_sc=False),
    out_type=(
        jax.ShapeDtypeStruct((_BATCH, _EMB), jnp.float32),
        jax.ShapeDtypeStruct((_BATCH, _EMB), jnp.float32),
    ),
    scratch_types=[
        pltpu.VMEM((_NCH, _CH), jnp.int32),
        pltpu.VMEM((_NCH, _CH), jnp.int32),
        pltpu.VMEM((_BPW, _EMB), jnp.float32),
        pltpu.VMEM((_BPW, _EMB), jnp.float32),
        pltpu.SemaphoreType.DMA,
        pltpu.SemaphoreType.DMA,
    ],
)
def _sc_dual_gather(tidx_hbm, cidx_hbm, ttab_hbm, ctab_hbm,
                    tout_hbm, cout_hbm,
                    tidx_v, cidx_v, trows_v, crows_v, tsem, csem):
    wid = lax.axis_index("s") * _NC + lax.axis_index("c")
    base = wid * _BPW
    pltpu.sync_copy(tidx_hbm.at[pl.ds(wid * _NCH, _NCH)], tidx_v)
    pltpu.sync_copy(cidx_hbm.at[pl.ds(wid * _NCH, _NCH)], cidx_v)
    handles = []
    for j in range(_NCH):
        handles.append(pltpu.async_copy(
            ttab_hbm.at[tidx_v.at[j]], trows_v.at[pl.ds(j * _CH, _CH)], tsem))
        handles.append(pltpu.async_copy(
            ctab_hbm.at[cidx_v.at[j]], crows_v.at[pl.ds(j * _CH, _CH)], csem))
    for h in handles:
        h.wait()
    pltpu.sync_copy(trows_v, tout_hbm.at[pl.ds(base, _BPW)])
    pltpu.sync_copy(crows_v, cout_hbm.at[pl.ds(base, _BPW)])


_BLK = 2048


def _ffnn_body(t_ref, c_ref, w1_ref, b1_ref, w2_ref, b2_ref, w3_ref, b3_ref,
               o_ref):
    w1 = w1_ref[...]
    h = jnp.dot(t_ref[...], w1[:_EMB], preferred_element_type=jnp.float32)
    h = h + jnp.dot(c_ref[...], w1[_EMB:], preferred_element_type=jnp.float32)
    h = h + b1_ref[...]
    h = jnp.dot(h, w2_ref[...], preferred_element_type=jnp.float32)
    h = h + b2_ref[...]
    o = jnp.dot(h, w3_ref[...], preferred_element_type=jnp.float32)
    o = o + b3_ref[...]
    o_ref[...] = jax.nn.sigmoid(o)


_ffnn_call = pl.pallas_call(
    _ffnn_body,
    grid=(_BATCH // _BLK,),
    in_specs=[
        pl.BlockSpec((_BLK, _EMB), lambda i: (i, 0)),
        pl.BlockSpec((_BLK, _EMB), lambda i: (i, 0)),
        pl.BlockSpec((2 * _EMB, _NODE), lambda i: (0, 0)),
        pl.BlockSpec((1, _NODE), lambda i: (0, 0)),
        pl.BlockSpec((_NODE, _NODE), lambda i: (0, 0)),
        pl.BlockSpec((1, _NODE), lambda i: (0, 0)),
        pl.BlockSpec((_NODE, 1), lambda i: (0, 0)),
        pl.BlockSpec((1, 1), lambda i: (0, 0)),
    ],
    out_specs=pl.BlockSpec((_BLK, 1), lambda i: (i, 0)),
    out_shape=jax.ShapeDtypeStruct((_BATCH, 1), jnp.float32),
)


@jax.jit
def kernel(inputs, target_table, context_table, W1, b1, W2, b2, W3, b3):
    idx = inputs.astype(jnp.int32)
    tidx = idx[:, 0].reshape(_NW * _NCH, _CH)
    cidx = idx[:, 1].reshape(_NW * _NCH, _CH)
    t_rows, c_rows = _sc_dual_gather(tidx, cidx, target_table, context_table)
    return _ffnn_call(
        t_rows, c_rows, W1, b1.reshape(1, _NODE), W2, b2.reshape(1, _NODE),
        W3, b3.reshape(1, 1))
